# baseline (speedup 1.0000x reference)
import functools

import jax
import jax.numpy as jnp
from jax import lax
from jax.experimental import pallas as pl
from jax.experimental.pallas import tpu as pltpu
from jax.experimental.pallas import tpu_sc as plsc

N = 10000
E = 160000
D = 256
HH = 128
B = 64
NC = 2
NS = 16
ROW_P = 624
ROW_LAST = N - ROW_P * (NS - 1)
INV_STD = 1.0 / (1.0 + 1e-5) ** 0.5


def _row_split_copy(s, copy_fn):
    @pl.when(s < NS - 1)
    def _():
        copy_fn(s * ROW_P, ROW_P)

    @pl.when(s == NS - 1)
    def _():
        copy_fn(ROW_P * (NS - 1), ROW_LAST)

DEG_W = 128
DEG_K = 40
DEG_EPW = E // (NC * NS)
DEG_T = DEG_EPW // DEG_K

AGG_K = 112
AGG_T = 93
AGG_EPS = AGG_K * AGG_T
AGG_PAD = AGG_EPS - E // NS
N_DUMMY = 16
ANB = 3
AGG_TM = AGG_T

_mesh = plsc.VectorSubcoreMesh(core_axis_name="c", subcore_axis_name="s")


NBUF = 5


@functools.partial(
    pl.kernel,
    out_type=[jax.ShapeDtypeStruct((N, DEG_W), jnp.float32),
              jax.ShapeDtypeStruct((N, DEG_W), jnp.float32)],
    mesh=_mesh,
    scratch_types=[
        pltpu.VMEM((NBUF, DEG_K), jnp.int32),
        pltpu.VMEM((DEG_K, DEG_W), jnp.float32),
        pltpu.VMEM_SHARED((N, DEG_W), jnp.float32),
    ] + [pltpu.SemaphoreType.DMA] * (2 * NBUF),
)
def _sc_degree(dst_hbm, zeros_hbm, ones_hbm, out0, out1, didx, ones_v, acc,
               *sems):
    sem_i = sems[:NBUF]
    sem_s = sems[NBUF:]
    c = lax.axis_index("c")
    s = lax.axis_index("s")
    _row_split_copy(s, lambda r0, nr: pltpu.sync_copy(
        zeros_hbm.at[pl.ds(r0, nr)], acc.at[pl.ds(r0, nr)]))
    pltpu.sync_copy(ones_hbm, ones_v)
    plsc.subcore_barrier()
    e0 = (s * NC + c) * DEG_EPW

    def issue_idx(t, b):
        pltpu.async_copy(dst_hbm.at[pl.ds(e0 + t * DEG_K, DEG_K)],
                         didx.at[b], sem_i[b])

    def wait_idx(b):
        pltpu.make_async_copy(dst_hbm.at[pl.ds(0, DEG_K)], didx.at[b],
                              sem_i[b]).wait()

    def issue_scatter(b):
        pltpu.async_copy(ones_v, acc.at[didx.at[b]], sem_s[b], add=True)

    def wait_scatter(b):
        pltpu.make_async_copy(ones_v, acc.at[didx.at[b]], sem_s[b]).wait()

    for b in range(NBUF):
        issue_idx(b, b)

    def outer(i, carry):
        g = i * NBUF
        for b in range(NBUF):
            wait_idx(b)
            issue_scatter(b)
        for b in range(NBUF):
            wait_scatter(b)
            issue_idx(g + NBUF + b, b)
        return carry

    lax.fori_loop(0, DEG_T // NBUF - 1, outer, 0)
    for b in range(NBUF):
        wait_idx(b)
        issue_scatter(b)
    for b in range(NBUF):
        wait_scatter(b)
    plsc.subcore_barrier()

    @pl.when(c == 0)
    def _():
        _row_split_copy(s, lambda r0, nr: pltpu.sync_copy(
            acc.at[pl.ds(r0, nr)], out0.at[pl.ds(r0, nr)]))

    @pl.when(c == 1)
    def _():
        _row_split_copy(s, lambda r0, nr: pltpu.sync_copy(
            acc.at[pl.ds(r0, nr)], out1.at[pl.ds(r0, nr)]))


@functools.partial(
    pl.kernel,
    out_type=[jax.ShapeDtypeStruct((N, HH), jnp.float32),
              jax.ShapeDtypeStruct((N, HH), jnp.float32)],
    mesh=_mesh,
    scratch_types=[
        pltpu.VMEM((ANB, AGG_K), jnp.int32),
        pltpu.VMEM((ANB, AGG_K), jnp.int32),
        pltpu.VMEM((ANB, AGG_K, HH), jnp.float32),
        pltpu.VMEM_SHARED((N + N_DUMMY, HH), jnp.float32),
    ] + [pltpu.SemaphoreType.DMA] * (3 * ANB),
)
def _sc_aggregate(hsl_hbm, hsr_hbm, src_hbm, dst_hbm, outl, outr,
                  sidx, didx, rows, acc, *sems):
    sem_g = sems[:ANB]
    sem_s = sems[ANB:2 * ANB]
    sem_i = sems[2 * ANB:]
    c = lax.axis_index("c")
    s = lax.axis_index("s")
    e0 = s * AGG_EPS

    def half(hs_hbm, out_hbm):
        _row_split_copy(s, lambda r0, nr: pltpu.sync_copy(
            hs_hbm.at[pl.ds(r0, nr)], acc.at[pl.ds(r0, nr)]))
        plsc.subcore_barrier()

        def issue_idx(t, b):
            base = e0 + t * AGG_K
            pltpu.async_copy(src_hbm.at[pl.ds(base, AGG_K)],
                             sidx.at[b], sem_i[b])
            pltpu.async_copy(dst_hbm.at[pl.ds(base, AGG_K)],
                             didx.at[b], sem_i[b])

        def wait_idx(b):
            pltpu.make_async_copy(src_hbm.at[pl.ds(0, AGG_K)], sidx.at[b],
                                  sem_i[b]).wait()
            pltpu.make_async_copy(dst_hbm.at[pl.ds(0, AGG_K)], didx.at[b],
                                  sem_i[b]).wait()

        def issue_gather(b):
            pltpu.async_copy(hs_hbm.at[sidx.at[b]], rows.at[b], sem_g[b])

        def wait_gather(b):
            pltpu.make_async_copy(hs_hbm.at[sidx.at[b]], rows.at[b],
                                  sem_g[b]).wait()

        def issue_scatter(b):
            pltpu.async_copy(rows.at[b], acc.at[didx.at[b]], sem_s[b],
                             add=True)

        def wait_scatter(b):
            pltpu.make_async_copy(rows.at[b], acc.at[didx.at[b]],
                                  sem_s[b]).wait()

        for b in range(ANB):
            issue_idx(b, b)

        def outer(i, carry):
            g = i * ANB
            for b in range(ANB):
                wait_idx(b)
                issue_gather(b)
            for b in range(ANB):
                wait_gather(b)
                issue_scatter(b)
            for b in range(ANB):
                wait_scatter(b)
                issue_idx(g + ANB + b, b)
            return carry

        lax.fori_loop(0, AGG_TM // ANB - 1, outer, 0)
        for b in range(ANB):
            wait_idx(b)
            issue_gather(b)
        for b in range(ANB):
            wait_gather(b)
            issue_scatter(b)
        for b in range(ANB):
            wait_scatter(b)
        plsc.subcore_barrier()
        _row_split_copy(s, lambda r0, nr: pltpu.sync_copy(
            acc.at[pl.ds(r0, nr)], out_hbm.at[pl.ds(r0, nr)]))

    @pl.when(c == 0)
    def _():
        half(hsl_hbm, outl)

    @pl.when(c == 1)
    def _():
        half(hsr_hbm, outr)


_BM = 2000


def _prep_body(deg0_ref, deg1_ref, x_ref, w_ref, dinv_ref, hsl_ref, hsr_ref):
    deg = deg0_ref[:, 0:1] + deg1_ref[:, 0:1] + 1.0
    dinv = lax.rsqrt(deg)
    hs = jnp.dot(x_ref[...], w_ref[...],
                 preferred_element_type=jnp.float32) * dinv
    dinv_ref[...] = jnp.broadcast_to(dinv, dinv_ref.shape)
    hsl_ref[...] = hs[:, :HH]
    hsr_ref[...] = hs[:, HH:]


def _tc_prep(deg0, deg1, x, w0):
    return pl.pallas_call(
        _prep_body,
        grid=(N // _BM,),
        in_specs=[
            pl.BlockSpec((_BM, DEG_W), lambda i: (i, 0)),
            pl.BlockSpec((_BM, DEG_W), lambda i: (i, 0)),
            pl.BlockSpec((_BM, D), lambda i: (i, 0)),
            pl.BlockSpec((D, D), lambda i: (0, 0)),
        ],
        out_specs=[
            pl.BlockSpec((_BM, DEG_W), lambda i: (i, 0)),
            pl.BlockSpec((_BM, HH), lambda i: (i, 0)),
            pl.BlockSpec((_BM, HH), lambda i: (i, 0)),
        ],
        out_shape=[
            jax.ShapeDtypeStruct((N, DEG_W), jnp.float32),
            jax.ShapeDtypeStruct((N, HH), jnp.float32),
            jax.ShapeDtypeStruct((N, HH), jnp.float32),
        ],
    )(deg0, deg1, x, w0)


def _post_body(has_res, has_next, aggl_ref, aggr_ref, dinv_ref, b_ref, g_ref,
               be_ref, *rest):
    if has_res:
        r_ref = rest[0]
        rest = rest[1:]
    if has_next:
        w_ref = rest[0]
        rest = rest[1:]
    dinv = dinv_ref[:, 0:1]
    agg = jnp.concatenate([aggl_ref[...], aggr_ref[...]], axis=1)
    conv = agg * dinv + b_ref[...]
    h = jnp.maximum(conv * INV_STD * g_ref[...] + be_ref[...], 0.0)
    if has_res:
        h = h + r_ref[...]
    if has_next:
        h_ref, hsl_ref, hsr_ref = rest
        hs = jnp.dot(h, w_ref[...], preferred_element_type=jnp.float32) * dinv
        h_ref[...] = h
        hsl_ref[...] = hs[:, :HH]
        hsr_ref[...] = hs[:, HH:]
    else:
        rest[0][...] = h


def _tc_post(aggl, aggr, dinv, b, g, be, r=None, w_next=None):
    has_res = r is not None
    has_next = w_next is not None
    row = pl.BlockSpec((_BM, D), lambda i: (i, 0))
    half = pl.BlockSpec((_BM, HH), lambda i: (i, 0))
    vec = pl.BlockSpec((1, D), lambda i: (0, 0))
    in_specs = [half, half, pl.BlockSpec((_BM, DEG_W), lambda i: (i, 0)),
                vec, vec, vec]
    args = [aggl, aggr, dinv, b.reshape(1, D), g.reshape(1, D),
            be.reshape(1, D)]
    if has_res:
        in_specs.append(row)
        args.append(r)
    if has_next:
        in_specs.append(pl.BlockSpec((D, D), lambda i: (0, 0)))
        args.append(w_next)
        out_specs = [row, half, half]
        out_shape = [jax.ShapeDtypeStruct((N, D), jnp.float32),
                     jax.ShapeDtypeStruct((N, HH), jnp.float32),
                     jax.ShapeDtypeStruct((N, HH), jnp.float32)]
    else:
        out_specs = [row]
        out_shape = [jax.ShapeDtypeStruct((N, D), jnp.float32)]
    return pl.pallas_call(
        functools.partial(_post_body, has_res, has_next),
        grid=(N // _BM,),
        in_specs=in_specs,
        out_specs=out_specs,
        out_shape=out_shape,
    )(*args)


def _post_head_body(aggl_ref, aggr_ref, dinv_ref, b_ref, g_ref, be_ref,
                    r_ref, batch_ref, aw1_ref, ab1_ref, aw2_ref, ab2_ref,
                    pw1_ref, pb1_ref, pw2_ref, pb2_ref, pw3_ref, pb3_ref,
                    out_ref, sums_ref):
    i = pl.program_id(0)
    dinv = dinv_ref[:, 0:1]
    agg = jnp.concatenate([aggl_ref[...], aggr_ref[...]], axis=1)
    conv = agg * dinv + b_ref[...]
    h = jnp.maximum(conv * INV_STD * g_ref[...] + be_ref[...], 0.0) \
        + r_ref[...]
    gid = lax.broadcasted_iota(jnp.int32, (1, B), 1)
    oh = (batch_ref[...] == gid).astype(jnp.float32)
    hx = jnp.concatenate([h, jnp.ones((h.shape[0], HH), jnp.float32)], 1)
    psum = lax.dot_general(oh, hx, (((0,), (0,)), ((), ())),
                           preferred_element_type=jnp.float32)

    @pl.when(i == 0)
    def _():
        sums_ref[...] = psum

    @pl.when(i > 0)
    def _():
        sums_ref[...] += psum

    @pl.when(i == N // _BM - 1)
    def _():
        sums = sums_ref[...]
        pooled = sums[:, :D] / jnp.maximum(sums[:, D:D + 1], 1.0)

        def dense(t, w_ref, bias_ref, act):
            y = jnp.dot(t, w_ref[...], preferred_element_type=jnp.float32) \
                + bias_ref[...]
            if act == "relu":
                return jnp.maximum(y, 0.0)
            return 1.0 / (1.0 + jnp.exp(-y))

        t = dense(pooled, aw1_ref, ab1_ref, "relu")
        t = dense(t, aw2_ref, ab2_ref, "relu")
        t = dense(t, pw1_ref, pb1_ref, "relu")
        t = dense(t, pw2_ref, pb2_ref, "relu")
        out_ref[...] = dense(t, pw3_ref, pb3_ref, "sigmoid")


def _tc_post_head(aggl, aggr, dinv, b, g, be, r, batch2d, aw1, ab1, aw2,
                  ab2, pw1, pb1, pw2, pb2, pw3, pb3):
    row = pl.BlockSpec((_BM, D), lambda i: (i, 0))
    half = pl.BlockSpec((_BM, HH), lambda i: (i, 0))
    vec = pl.BlockSpec((1, D), lambda i: (0, 0))

    def full(a):
        return pl.BlockSpec(a.shape, lambda i: tuple(0 for _ in a.shape))

    args = [aggl, aggr, dinv, b.reshape(1, D), g.reshape(1, D),
            be.reshape(1, D), r, batch2d, aw1, ab1.reshape(1, -1),
            aw2, ab2.reshape(1, -1), pw1, pb1.reshape(1, -1),
            pw2, pb2.reshape(1, -1), pw3, pb3.reshape(1, -1)]
    in_specs = [half, half, pl.BlockSpec((_BM, DEG_W), lambda i: (i, 0)),
                vec, vec, vec, row, pl.BlockSpec((_BM, 1), lambda i: (i, 0))]
    in_specs += [full(a) for a in args[8:]]
    return pl.pallas_call(
        _post_head_body,
        grid=(N // _BM,),
        in_specs=in_specs,
        out_specs=pl.BlockSpec((B, 1), lambda i: (0, 0)),
        out_shape=jax.ShapeDtypeStruct((B, 1), jnp.float32),
        scratch_shapes=[pltpu.VMEM((B, D + HH), jnp.float32)],
    )(*args)


def kernel(x, edge_index, batch, W0, b0, W1, b1, W2, b2, g0, be0, g1, be1,
           g2, be2, aw1, ab1, aw2, ab2, pw1, pb1, pw2, pb2, pw3, pb3):
    src = edge_index[0]
    dst = edge_index[1]
    epw = E // NS
    pad_src = jnp.zeros((NS, AGG_PAD), jnp.int32)
    pad_dst = jnp.broadcast_to(
        N + (jnp.arange(AGG_PAD, dtype=jnp.int32) % N_DUMMY),
        (NS, AGG_PAD))
    src_p = jnp.concatenate([src.reshape(NS, epw), pad_src], 1).reshape(-1)
    dst_p = jnp.concatenate([dst.reshape(NS, epw), pad_dst], 1).reshape(-1)
    zeros_nw = jnp.zeros((N, DEG_W), jnp.float32)
    ones_kw = jnp.ones((DEG_K, DEG_W), jnp.float32)

    deg0, deg1 = _sc_degree(dst, zeros_nw, ones_kw)
    dinv, hs0l, hs0r = _tc_prep(deg0, deg1, x, W0)

    agg0l, agg0r = _sc_aggregate(hs0l, hs0r, src_p, dst_p)
    h1, hs1l, hs1r = _tc_post(agg0l, agg0r, dinv, b0, g0, be0, w_next=W1)

    agg1l, agg1r = _sc_aggregate(hs1l, hs1r, src_p, dst_p)
    h2, hs2l, hs2r = _tc_post(agg1l, agg1r, dinv, b1, g1, be1, r=h1,
                              w_next=W2)

    agg2l, agg2r = _sc_aggregate(hs2l, hs2r, src_p, dst_p)
    out = _tc_post_head(agg2l, agg2r, dinv, b2, g2, be2, h2,
                        batch.reshape(N, 1), aw1, ab1, aw2, ab2,
                        pw1, pb1, pw2, pb2, pw3, pb3)
    return out.reshape(B)


# speedup vs baseline: 2.3684x; 2.3684x over previous
import functools

import jax
import jax.numpy as jnp
from jax import lax
from jax.experimental import pallas as pl
from jax.experimental.pallas import tpu as pltpu
from jax.experimental.pallas import tpu_sc as plsc

N = 10000
E = 160000
D = 256
HH = 128
B = 64
NC = 2
NS = 16
ROW_P = 624
ROW_LAST = N - ROW_P * (NS - 1)
INV_STD = 1.0 / (1.0 + 1e-5) ** 0.5


def _row_split_copy(s, copy_fn):
    @pl.when(s < NS - 1)
    def _():
        copy_fn(s * ROW_P, ROW_P)

    @pl.when(s == NS - 1)
    def _():
        copy_fn(ROW_P * (NS - 1), ROW_LAST)

DEG_W = 128
DEG_K = 40
DEG_EPW = E // (NC * NS)
DEG_T = DEG_EPW // DEG_K

AGG_K = 80
AGG_EPS = E // NS
AGG_T = AGG_EPS // AGG_K
ANB = 4
AGG_TM = AGG_T - (AGG_T % ANB)

_mesh = plsc.VectorSubcoreMesh(core_axis_name="c", subcore_axis_name="s")


NBUF = 5


@functools.partial(
    pl.kernel,
    out_type=[jax.ShapeDtypeStruct((N, DEG_W), jnp.float32),
              jax.ShapeDtypeStruct((N, DEG_W), jnp.float32)],
    mesh=_mesh,
    scratch_types=[
        pltpu.VMEM((NBUF, DEG_K), jnp.int32),
        pltpu.VMEM((DEG_K, DEG_W), jnp.float32),
        pltpu.VMEM_SHARED((N, DEG_W), jnp.float32),
    ] + [pltpu.SemaphoreType.DMA] * (2 * NBUF),
)
def _sc_degree(dst_hbm, zeros_hbm, ones_hbm, out0, out1, didx, ones_v, acc,
               *sems):
    sem_i = sems[:NBUF]
    sem_s = sems[NBUF:]
    c = lax.axis_index("c")
    s = lax.axis_index("s")
    _row_split_copy(s, lambda r0, nr: pltpu.sync_copy(
        zeros_hbm.at[pl.ds(r0, nr)], acc.at[pl.ds(r0, nr)]))
    pltpu.sync_copy(ones_hbm, ones_v)
    plsc.subcore_barrier()
    e0 = (s * NC + c) * DEG_EPW

    def issue_idx(t, b):
        pltpu.async_copy(dst_hbm.at[pl.ds(e0 + t * DEG_K, DEG_K)],
                         didx.at[b], sem_i[b])

    def wait_idx(b):
        pltpu.make_async_copy(dst_hbm.at[pl.ds(0, DEG_K)], didx.at[b],
                              sem_i[b]).wait()

    def issue_scatter(b):
        pltpu.async_copy(ones_v, acc.at[didx.at[b]], sem_s[b], add=True)

    def wait_scatter(b):
        pltpu.make_async_copy(ones_v, acc.at[didx.at[b]], sem_s[b]).wait()

    for b in range(NBUF):
        issue_idx(b, b)

    def outer(i, carry):
        g = i * NBUF
        for b in range(NBUF):
            wait_idx(b)
            issue_scatter(b)
        for b in range(NBUF):
            wait_scatter(b)
            issue_idx(g + NBUF + b, b)
        return carry

    lax.fori_loop(0, DEG_T // NBUF - 1, outer, 0)
    for b in range(NBUF):
        wait_idx(b)
        issue_scatter(b)
    for b in range(NBUF):
        wait_scatter(b)
    plsc.subcore_barrier()

    @pl.when(c == 0)
    def _():
        _row_split_copy(s, lambda r0, nr: pltpu.sync_copy(
            acc.at[pl.ds(r0, nr)], out0.at[pl.ds(r0, nr)]))

    @pl.when(c == 1)
    def _():
        _row_split_copy(s, lambda r0, nr: pltpu.sync_copy(
            acc.at[pl.ds(r0, nr)], out1.at[pl.ds(r0, nr)]))


@functools.partial(
    pl.kernel,
    out_type=[jax.ShapeDtypeStruct((N, HH), jnp.float32),
              jax.ShapeDtypeStruct((N, HH), jnp.float32)],
    mesh=_mesh,
    scratch_types=[
        pltpu.VMEM((ANB, AGG_K), jnp.int32),
        pltpu.VMEM((ANB, AGG_K), jnp.int32),
        pltpu.VMEM((ANB, AGG_K, HH), jnp.float32),
        pltpu.VMEM_SHARED((N, HH), jnp.float32),
    ] + [pltpu.SemaphoreType.DMA] * (3 * ANB),
)
def _sc_aggregate(hsl_hbm, hsr_hbm, src_hbm, dst_hbm, outl, outr,
                  sidx, didx, rows, acc, *sems):
    sem_g = sems[:ANB]
    sem_s = sems[ANB:2 * ANB]
    sem_i = sems[2 * ANB:]
    c = lax.axis_index("c")
    s = lax.axis_index("s")
    e0 = s * AGG_EPS

    def half(hs_hbm, out_hbm):
        _row_split_copy(s, lambda r0, nr: pltpu.sync_copy(
            hs_hbm.at[pl.ds(r0, nr)], acc.at[pl.ds(r0, nr)]))
        plsc.subcore_barrier()

        def issue_idx(t, b):
            base = e0 + t * AGG_K
            pltpu.async_copy(src_hbm.at[pl.ds(base, AGG_K)],
                             sidx.at[b], sem_i[b])
            pltpu.async_copy(dst_hbm.at[pl.ds(base, AGG_K)],
                             didx.at[b], sem_i[b])

        def wait_idx(b):
            pltpu.make_async_copy(src_hbm.at[pl.ds(0, AGG_K)], sidx.at[b],
                                  sem_i[b]).wait()
            pltpu.make_async_copy(dst_hbm.at[pl.ds(0, AGG_K)], didx.at[b],
                                  sem_i[b]).wait()

        def issue_gather(b):
            pltpu.async_copy(hs_hbm.at[sidx.at[b]], rows.at[b], sem_g[b])

        def wait_gather(b):
            pltpu.make_async_copy(hs_hbm.at[sidx.at[b]], rows.at[b],
                                  sem_g[b]).wait()

        def issue_scatter(b):
            pltpu.async_copy(rows.at[b], acc.at[didx.at[b]], sem_s[b],
                             add=True)

        def wait_scatter(b):
            pltpu.make_async_copy(rows.at[b], acc.at[didx.at[b]],
                                  sem_s[b]).wait()

        for b in range(ANB):
            issue_idx(b, b)

        def outer(i, carry):
            g = i * ANB
            for b in range(ANB):
                wait_idx(b)
                issue_gather(b)
            for b in range(ANB):
                wait_gather(b)
                issue_scatter(b)
            for b in range(ANB):
                wait_scatter(b)
                issue_idx(g + ANB + b, b)
            return carry

        lax.fori_loop(0, AGG_TM // ANB - 1, outer, 0)
        for b in range(ANB):
            wait_idx(b)
            issue_gather(b)
        for b in range(ANB):
            wait_gather(b)
            issue_scatter(b)
        for b in range(ANB):
            wait_scatter(b)
        issue_idx(AGG_T - 1, 0)
        wait_idx(0)
        issue_gather(0)
        wait_gather(0)
        issue_scatter(0)
        wait_scatter(0)
        plsc.subcore_barrier()
        _row_split_copy(s, lambda r0, nr: pltpu.sync_copy(
            acc.at[pl.ds(r0, nr)], out_hbm.at[pl.ds(r0, nr)]))

    @pl.when(c == 0)
    def _():
        half(hsl_hbm, outl)

    @pl.when(c == 1)
    def _():
        half(hsr_hbm, outr)


_BM = 2000


def _prep_body(deg0_ref, deg1_ref, x_ref, w_ref, dinv_ref, hsl_ref, hsr_ref):
    deg = deg0_ref[:, 0:1] + deg1_ref[:, 0:1] + 1.0
    dinv = lax.rsqrt(deg)
    hs = jnp.dot(x_ref[...], w_ref[...],
                 preferred_element_type=jnp.float32) * dinv
    dinv_ref[...] = jnp.broadcast_to(dinv, dinv_ref.shape)
    hsl_ref[...] = hs[:, :HH]
    hsr_ref[...] = hs[:, HH:]


def _tc_prep(deg0, deg1, x, w0):
    return pl.pallas_call(
        _prep_body,
        grid=(N // _BM,),
        in_specs=[
            pl.BlockSpec((_BM, DEG_W), lambda i: (i, 0)),
            pl.BlockSpec((_BM, DEG_W), lambda i: (i, 0)),
            pl.BlockSpec((_BM, D), lambda i: (i, 0)),
            pl.BlockSpec((D, D), lambda i: (0, 0)),
        ],
        out_specs=[
            pl.BlockSpec((_BM, DEG_W), lambda i: (i, 0)),
            pl.BlockSpec((_BM, HH), lambda i: (i, 0)),
            pl.BlockSpec((_BM, HH), lambda i: (i, 0)),
        ],
        out_shape=[
            jax.ShapeDtypeStruct((N, DEG_W), jnp.float32),
            jax.ShapeDtypeStruct((N, HH), jnp.float32),
            jax.ShapeDtypeStruct((N, HH), jnp.float32),
        ],
    )(deg0, deg1, x, w0)


def _post_body(has_res, has_next, aggl_ref, aggr_ref, dinv_ref, b_ref, g_ref,
               be_ref, *rest):
    if has_res:
        r_ref = rest[0]
        rest = rest[1:]
    if has_next:
        w_ref = rest[0]
        rest = rest[1:]
    dinv = dinv_ref[:, 0:1]
    agg = jnp.concatenate([aggl_ref[...], aggr_ref[...]], axis=1)
    conv = agg * dinv + b_ref[...]
    h = jnp.maximum(conv * INV_STD * g_ref[...] + be_ref[...], 0.0)
    if has_res:
        h = h + r_ref[...]
    if has_next:
        h_ref, hsl_ref, hsr_ref = rest
        hs = jnp.dot(h, w_ref[...], preferred_element_type=jnp.float32) * dinv
        h_ref[...] = h
        hsl_ref[...] = hs[:, :HH]
        hsr_ref[...] = hs[:, HH:]
    else:
        rest[0][...] = h


def _tc_post(aggl, aggr, dinv, b, g, be, r=None, w_next=None):
    has_res = r is not None
    has_next = w_next is not None
    row = pl.BlockSpec((_BM, D), lambda i: (i, 0))
    half = pl.BlockSpec((_BM, HH), lambda i: (i, 0))
    vec = pl.BlockSpec((1, D), lambda i: (0, 0))
    in_specs = [half, half, pl.BlockSpec((_BM, DEG_W), lambda i: (i, 0)),
                vec, vec, vec]
    args = [aggl, aggr, dinv, b.reshape(1, D), g.reshape(1, D),
            be.reshape(1, D)]
    if has_res:
        in_specs.append(row)
        args.append(r)
    if has_next:
        in_specs.append(pl.BlockSpec((D, D), lambda i: (0, 0)))
        args.append(w_next)
        out_specs = [row, half, half]
        out_shape = [jax.ShapeDtypeStruct((N, D), jnp.float32),
                     jax.ShapeDtypeStruct((N, HH), jnp.float32),
                     jax.ShapeDtypeStruct((N, HH), jnp.float32)]
    else:
        out_specs = [row]
        out_shape = [jax.ShapeDtypeStruct((N, D), jnp.float32)]
    return pl.pallas_call(
        functools.partial(_post_body, has_res, has_next),
        grid=(N // _BM,),
        in_specs=in_specs,
        out_specs=out_specs,
        out_shape=out_shape,
    )(*args)


def _post_head_body(aggl_ref, aggr_ref, dinv_ref, b_ref, g_ref, be_ref,
                    r_ref, batch_ref, aw1_ref, ab1_ref, aw2_ref, ab2_ref,
                    pw1_ref, pb1_ref, pw2_ref, pb2_ref, pw3_ref, pb3_ref,
                    out_ref, sums_ref):
    i = pl.program_id(0)
    dinv = dinv_ref[:, 0:1]
    agg = jnp.concatenate([aggl_ref[...], aggr_ref[...]], axis=1)
    conv = agg * dinv + b_ref[...]
    h = jnp.maximum(conv * INV_STD * g_ref[...] + be_ref[...], 0.0) \
        + r_ref[...]
    gid = lax.broadcasted_iota(jnp.int32, (1, B), 1)
    oh = (batch_ref[...] == gid).astype(jnp.float32)
    hx = jnp.concatenate([h, jnp.ones((h.shape[0], HH), jnp.float32)], 1)
    psum = lax.dot_general(oh, hx, (((0,), (0,)), ((), ())),
                           preferred_element_type=jnp.float32)

    @pl.when(i == 0)
    def _():
        sums_ref[...] = psum

    @pl.when(i > 0)
    def _():
        sums_ref[...] += psum

    @pl.when(i == N // _BM - 1)
    def _():
        sums = sums_ref[...]
        pooled = sums[:, :D] / jnp.maximum(sums[:, D:D + 1], 1.0)

        def dense(t, w_ref, bias_ref, act):
            y = jnp.dot(t, w_ref[...], preferred_element_type=jnp.float32) \
                + bias_ref[...]
            if act == "relu":
                return jnp.maximum(y, 0.0)
            return 1.0 / (1.0 + jnp.exp(-y))

        t = dense(pooled, aw1_ref, ab1_ref, "relu")
        t = dense(t, aw2_ref, ab2_ref, "relu")
        t = dense(t, pw1_ref, pb1_ref, "relu")
        t = dense(t, pw2_ref, pb2_ref, "relu")
        out_ref[...] = dense(t, pw3_ref, pb3_ref, "sigmoid")


def _tc_post_head(aggl, aggr, dinv, b, g, be, r, batch2d, aw1, ab1, aw2,
                  ab2, pw1, pb1, pw2, pb2, pw3, pb3):
    row = pl.BlockSpec((_BM, D), lambda i: (i, 0))
    half = pl.BlockSpec((_BM, HH), lambda i: (i, 0))
    vec = pl.BlockSpec((1, D), lambda i: (0, 0))

    def full(a):
        return pl.BlockSpec(a.shape, lambda i: tuple(0 for _ in a.shape))

    args = [aggl, aggr, dinv, b.reshape(1, D), g.reshape(1, D),
            be.reshape(1, D), r, batch2d, aw1, ab1.reshape(1, -1),
            aw2, ab2.reshape(1, -1), pw1, pb1.reshape(1, -1),
            pw2, pb2.reshape(1, -1), pw3, pb3.reshape(1, -1)]
    in_specs = [half, half, pl.BlockSpec((_BM, DEG_W), lambda i: (i, 0)),
                vec, vec, vec, row, pl.BlockSpec((_BM, 1), lambda i: (i, 0))]
    in_specs += [full(a) for a in args[8:]]
    return pl.pallas_call(
        _post_head_body,
        grid=(N // _BM,),
        in_specs=in_specs,
        out_specs=pl.BlockSpec((B, 1), lambda i: (0, 0)),
        out_shape=jax.ShapeDtypeStruct((B, 1), jnp.float32),
        scratch_shapes=[pltpu.VMEM((B, D + HH), jnp.float32)],
    )(*args)


def kernel(x, edge_index, batch, W0, b0, W1, b1, W2, b2, g0, be0, g1, be1,
           g2, be2, aw1, ab1, aw2, ab2, pw1, pb1, pw2, pb2, pw3, pb3):
    src = edge_index[0]
    dst = edge_index[1]
    zeros_nw = jnp.zeros((N, DEG_W), jnp.float32)
    ones_kw = jnp.ones((DEG_K, DEG_W), jnp.float32)

    deg0, deg1 = _sc_degree(dst, zeros_nw, ones_kw)
    dinv, hs0l, hs0r = _tc_prep(deg0, deg1, x, W0)

    agg0l, agg0r = _sc_aggregate(hs0l, hs0r, src, dst)
    h1, hs1l, hs1r = _tc_post(agg0l, agg0r, dinv, b0, g0, be0, w_next=W1)

    agg1l, agg1r = _sc_aggregate(hs1l, hs1r, src, dst)
    h2, hs2l, hs2r = _tc_post(agg1l, agg1r, dinv, b1, g1, be1, r=h1,
                              w_next=W2)

    agg2l, agg2r = _sc_aggregate(hs2l, hs2r, src, dst)
    out = _tc_post_head(agg2l, agg2r, dinv, b2, g2, be2, h2,
                        batch.reshape(N, 1), aw1, ab1, aw2, ab2,
                        pw1, pb1, pw2, pb2, pw3, pb3)
    return out.reshape(B)
